# Initial kernel scaffold; baseline (speedup 1.0000x reference)
import jax
import jax.numpy as jnp
from jax.experimental import pallas as pl


def kernel(hidden_states, weight):
    raise NotImplementedError("write your pallas kernel here")


# speedup vs baseline: 5.3578x; 5.3578x over previous
import functools

import jax
import jax.numpy as jnp
from jax import lax
from jax.experimental import pallas as pl
from jax.experimental.pallas import tpu as pltpu

E = 64
NG = 8
GSZ = E // NG
TOPKG = 3
K = 8
BLOCK_T = 512


def _gate_kernel(x_ref, w_ref, idx_ref, wgt_ref):
    T = x_ref.shape[0]
    logits = lax.dot_general(
        w_ref[...], x_ref[...],
        dimension_numbers=(((1,), (1,)), ((), ())),
        preferred_element_type=jnp.float32,
    )
    m = jnp.max(logits, axis=0, keepdims=True)
    unnorm = jnp.exp(logits - m)
    scores = unnorm / jnp.sum(unnorm, axis=0, keepdims=True)

    gmax = jnp.concatenate(
        [jnp.max(scores[g * GSZ:(g + 1) * GSZ, :], axis=0, keepdims=True)
         for g in range(NG)], axis=0)

    giota = lax.broadcasted_iota(jnp.int32, (NG, T), 0)
    keep_g = jnp.zeros((NG, T), dtype=jnp.bool_)
    avail_g = gmax
    for _ in range(TOPKG):
        gm = jnp.max(avail_g, axis=0, keepdims=True)
        gsel = jnp.min(jnp.where(avail_g == gm, giota, NG), axis=0,
                       keepdims=True)
        hit = giota == gsel
        keep_g = jnp.logical_or(keep_g, hit)
        avail_g = jnp.where(hit, -1.0, avail_g)

    keep_e = jnp.concatenate(
        [jnp.broadcast_to(keep_g[g:g + 1, :], (GSZ, T)) for g in range(NG)],
        axis=0)
    masked = jnp.where(keep_e, scores, 0.0)

    eiota = lax.broadcasted_iota(jnp.int32, (E, T), 0)
    avail = masked
    idx_rows = []
    val_rows = []
    for _ in range(K):
        mv = jnp.max(avail, axis=0, keepdims=True)
        sel = jnp.min(jnp.where(avail == mv, eiota, E), axis=0, keepdims=True)
        idx_rows.append(sel)
        val_rows.append(mv)
        avail = jnp.where(eiota == sel, -1.0, avail)
    topk_idx = jnp.concatenate(idx_rows, axis=0)
    topk_val = jnp.concatenate(val_rows, axis=0)

    denom = jnp.sum(topk_val, axis=0, keepdims=True) + 1e-20
    idx_ref[...] = topk_idx
    wgt_ref[...] = topk_val / denom


@jax.jit
def kernel(hidden_states, weight):
    bsz, seq, h = hidden_states.shape
    x = hidden_states.reshape(-1, h)
    n_tok = x.shape[0]
    grid = (n_tok // BLOCK_T,)
    idx_t, wgt_t = pl.pallas_call(
        _gate_kernel,
        grid=grid,
        in_specs=[
            pl.BlockSpec((BLOCK_T, h), lambda i: (i, 0)),
            pl.BlockSpec((E, h), lambda i: (0, 0)),
        ],
        out_specs=[
            pl.BlockSpec((K, BLOCK_T), lambda i: (0, i)),
            pl.BlockSpec((K, BLOCK_T), lambda i: (0, i)),
        ],
        out_shape=[
            jax.ShapeDtypeStruct((K, n_tok), jnp.int32),
            jax.ShapeDtypeStruct((K, n_tok), jnp.float32),
        ],
        compiler_params=pltpu.CompilerParams(
            dimension_semantics=("arbitrary",),
        ),
    )(x, weight)
    return idx_t.T, wgt_t.T, None


# speedup vs baseline: 6.4979x; 1.2128x over previous
import functools

import jax
import jax.numpy as jnp
from jax import lax
from jax.experimental import pallas as pl
from jax.experimental.pallas import tpu as pltpu

E = 64
NG = 8
GSZ = E // NG
TOPKG = 3
K = 8
BLOCK_T = 1024


def _gate_kernel(x_ref, w_ref, idx_ref, wgt_ref):
    T = x_ref.shape[0]
    logits = lax.dot_general(
        w_ref[...], x_ref[...],
        dimension_numbers=(((1,), (1,)), ((), ())),
        preferred_element_type=jnp.float32,
    )
    m = jnp.max(logits, axis=0, keepdims=True)
    unnorm = jnp.exp(logits - m)
    scores = unnorm / jnp.sum(unnorm, axis=0, keepdims=True)

    gmax = jnp.concatenate(
        [jnp.max(scores[g * GSZ:(g + 1) * GSZ, :], axis=0, keepdims=True)
         for g in range(NG)], axis=0)

    giota = lax.broadcasted_iota(jnp.int32, (NG, T), 0)
    keep_g = jnp.zeros((NG, T), dtype=jnp.bool_)
    avail_g = gmax
    for _ in range(TOPKG):
        gm = jnp.max(avail_g, axis=0, keepdims=True)
        gsel = jnp.min(jnp.where(avail_g == gm, giota, NG), axis=0,
                       keepdims=True)
        hit = giota == gsel
        keep_g = jnp.logical_or(keep_g, hit)
        avail_g = jnp.where(hit, -1.0, avail_g)

    keep_e = jnp.concatenate(
        [jnp.broadcast_to(keep_g[g:g + 1, :], (GSZ, T)) for g in range(NG)],
        axis=0)
    masked = jnp.where(keep_e, scores, 0.0)

    eiota = lax.broadcasted_iota(jnp.int32, (E, T), 0)
    avail = masked
    idx_rows = []
    val_rows = []
    for _ in range(K):
        mv = jnp.max(avail, axis=0, keepdims=True)
        sel = jnp.min(jnp.where(avail == mv, eiota, E), axis=0, keepdims=True)
        idx_rows.append(sel)
        val_rows.append(mv)
        avail = jnp.where(eiota == sel, -1.0, avail)
    topk_idx = jnp.concatenate(idx_rows, axis=0)
    topk_val = jnp.concatenate(val_rows, axis=0)

    denom = jnp.sum(topk_val, axis=0, keepdims=True) + 1e-20
    idx_ref[...] = topk_idx
    wgt_ref[...] = topk_val / denom


@jax.jit
def kernel(hidden_states, weight):
    bsz, seq, h = hidden_states.shape
    x = hidden_states.reshape(-1, h)
    n_tok = x.shape[0]
    grid = (n_tok // BLOCK_T,)
    idx_t, wgt_t = pl.pallas_call(
        _gate_kernel,
        grid=grid,
        in_specs=[
            pl.BlockSpec((BLOCK_T, h), lambda i: (i, 0)),
            pl.BlockSpec((E, h), lambda i: (0, 0)),
        ],
        out_specs=[
            pl.BlockSpec((K, BLOCK_T), lambda i: (0, i)),
            pl.BlockSpec((K, BLOCK_T), lambda i: (0, i)),
        ],
        out_shape=[
            jax.ShapeDtypeStruct((K, n_tok), jnp.int32),
            jax.ShapeDtypeStruct((K, n_tok), jnp.float32),
        ],
        compiler_params=pltpu.CompilerParams(
            dimension_semantics=("arbitrary",),
        ),
    )(x, weight)
    return idx_t.T, wgt_t.T, None


# speedup vs baseline: 6.7980x; 1.0462x over previous
import functools

import jax
import jax.numpy as jnp
from jax import lax
from jax.experimental import pallas as pl
from jax.experimental.pallas import tpu as pltpu

E = 64
NG = 8
GSZ = E // NG
TOPKG = 3
K = 8
BLOCK_T = 2048


def _gate_kernel(x_ref, w_ref, idx_ref, wgt_ref):
    T = x_ref.shape[0]
    logits = lax.dot_general(
        w_ref[...], x_ref[...],
        dimension_numbers=(((1,), (1,)), ((), ())),
        preferred_element_type=jnp.float32,
    )
    m = jnp.max(logits, axis=0, keepdims=True)
    unnorm = jnp.exp(logits - m)
    scores = unnorm / jnp.sum(unnorm, axis=0, keepdims=True)

    gmax = jnp.concatenate(
        [jnp.max(scores[g * GSZ:(g + 1) * GSZ, :], axis=0, keepdims=True)
         for g in range(NG)], axis=0)

    giota = lax.broadcasted_iota(jnp.int32, (NG, T), 0)
    keep_g = jnp.zeros((NG, T), dtype=jnp.bool_)
    avail_g = gmax
    for _ in range(TOPKG):
        gm = jnp.max(avail_g, axis=0, keepdims=True)
        gsel = jnp.min(jnp.where(avail_g == gm, giota, NG), axis=0,
                       keepdims=True)
        hit = giota == gsel
        keep_g = jnp.logical_or(keep_g, hit)
        avail_g = jnp.where(hit, -1.0, avail_g)

    keep_e = jnp.concatenate(
        [jnp.broadcast_to(keep_g[g:g + 1, :], (GSZ, T)) for g in range(NG)],
        axis=0)
    masked = jnp.where(keep_e, scores, 0.0)

    eiota = lax.broadcasted_iota(jnp.int32, (E, T), 0)
    avail = masked
    idx_rows = []
    val_rows = []
    for _ in range(K):
        mv = jnp.max(avail, axis=0, keepdims=True)
        sel = jnp.min(jnp.where(avail == mv, eiota, E), axis=0, keepdims=True)
        idx_rows.append(sel)
        val_rows.append(mv)
        avail = jnp.where(eiota == sel, -1.0, avail)
    topk_idx = jnp.concatenate(idx_rows, axis=0)
    topk_val = jnp.concatenate(val_rows, axis=0)

    denom = jnp.sum(topk_val, axis=0, keepdims=True) + 1e-20
    idx_ref[...] = topk_idx
    wgt_ref[...] = topk_val / denom


@jax.jit
def kernel(hidden_states, weight):
    bsz, seq, h = hidden_states.shape
    x = hidden_states.reshape(-1, h)
    n_tok = x.shape[0]
    grid = (n_tok // BLOCK_T,)
    idx_t, wgt_t = pl.pallas_call(
        _gate_kernel,
        grid=grid,
        in_specs=[
            pl.BlockSpec((BLOCK_T, h), lambda i: (i, 0)),
            pl.BlockSpec((E, h), lambda i: (0, 0)),
        ],
        out_specs=[
            pl.BlockSpec((K, BLOCK_T), lambda i: (0, i)),
            pl.BlockSpec((K, BLOCK_T), lambda i: (0, i)),
        ],
        out_shape=[
            jax.ShapeDtypeStruct((K, n_tok), jnp.int32),
            jax.ShapeDtypeStruct((K, n_tok), jnp.float32),
        ],
        compiler_params=pltpu.CompilerParams(
            dimension_semantics=("arbitrary",),
        ),
    )(x, weight)
    return idx_t.T, wgt_t.T, None
